# baseline (speedup 1.0000x reference)
import functools

import jax
import jax.numpy as jnp
from jax import lax
from jax.experimental import pallas as pl
from jax.experimental.pallas import tpu as pltpu
from jax.experimental.pallas import tpu_sc as plsc

N_NODES = 10000
N_EDGES = 320000
D = 128

NC = 2
NS = 16
NW = NC * NS

CHUNK = 96
E_PER_W = N_EDGES // NW
CH = E_PER_W // CHUNK
TAIL = E_PER_W - CH * CHUNK
STAGES = 4
SLAB = CH // STAGES
NBUF = 3
ROWS_PER_TILE = 624
ROWS_REM = N_NODES - NS * ROWS_PER_TILE


def _sc_gather_scatter(feature, src_main, dst_main, src_tail, dst_tail, zeros):
  mesh = plsc.VectorSubcoreMesh(core_axis_name="c", subcore_axis_name="s")

  @functools.partial(
      pl.kernel,
      mesh=mesh,
      out_type=jax.ShapeDtypeStruct((NC * N_NODES, D), jnp.float32),
      scratch_types=[
          pltpu.VMEM((SLAB, CHUNK), jnp.int32),
          pltpu.VMEM((SLAB, CHUNK), jnp.int32),
          pltpu.VMEM((TAIL,), jnp.int32),
          pltpu.VMEM((TAIL,), jnp.int32),
          pltpu.VMEM((CHUNK, D), jnp.float32),
          pltpu.VMEM((CHUNK, D), jnp.float32),
          pltpu.VMEM((CHUNK, D), jnp.float32),
          pltpu.VMEM_SHARED((N_NODES, D), jnp.float32),
          pltpu.SemaphoreType.DMA,
          pltpu.SemaphoreType.DMA,
          pltpu.SemaphoreType.DMA,
          pltpu.SemaphoreType.DMA,
          pltpu.SemaphoreType.DMA,
          pltpu.SemaphoreType.DMA,
      ],
  )
  def k(feat_hbm, srcm_hbm, dstm_hbm, srct_hbm, dstt_hbm, zero_hbm, out_hbm,
        src_v, dst_v, srct_v, dstt_v, r0_v, r1_v, r2_v, acc_s,
        gsem0, gsem1, gsem2, ssem0, ssem1, ssem2):
    c = lax.axis_index("c")
    s = lax.axis_index("s")
    wid = s * NC + c

    bufs = (r0_v, r1_v, r2_v)
    gsems = (gsem0, gsem1, gsem2)
    ssems = (ssem0, ssem1, ssem2)

    pltpu.sync_copy(zero_hbm,
                    acc_s.at[pl.ds(s * ROWS_PER_TILE, ROWS_PER_TILE)])

    @pl.when(s == NS - 1)
    def _():
      pltpu.sync_copy(zero_hbm.at[pl.ds(0, ROWS_REM)],
                      acc_s.at[pl.ds(NS * ROWS_PER_TILE, ROWS_REM)])

    pltpu.sync_copy(srct_hbm.at[pl.ds(wid * TAIL, TAIL)], srct_v)
    pltpu.sync_copy(dstt_hbm.at[pl.ds(wid * TAIL, TAIL)], dstt_v)
    plsc.subcore_barrier()

    def start_gather(j, b):
      pltpu.async_copy(feat_hbm.at[src_v.at[j]], bufs[b], gsems[b])

    def wait_gather(j, b):
      pltpu.make_async_copy(feat_hbm.at[src_v.at[j]], bufs[b],
                            gsems[b]).wait()

    def start_scatter(j, b):
      pltpu.async_copy(bufs[b], acc_s.at[dst_v.at[j]], ssems[b], add=True)

    def wait_scatter(j, b):
      pltpu.make_async_copy(bufs[b], acc_s.at[dst_v.at[j]], ssems[b]).wait()

    def stage_body(st, carry):
      pltpu.sync_copy(srcm_hbm.at[wid * STAGES + st], src_v)
      pltpu.sync_copy(dstm_hbm.at[wid * STAGES + st], dst_v)
      start_gather(0, 0)
      start_gather(1, 1)

      def tri(i, carry2):
        for d in range(NBUF):
          j = NBUF * i + d
          t2 = (d + 2) % NBUF
          wait_gather(j, d)
          start_scatter(j, d)

          @pl.when(j >= 1)
          def _():
            wait_scatter(j - 1, t2)

          start_gather(j + 2, t2)
        return carry2

      lax.fori_loop(0, SLAB // NBUF, tri, 0)
      for j, t in ((SLAB - 2, (SLAB - 2) % NBUF), (SLAB - 1, (SLAB - 1) % NBUF)):
        wait_gather(jnp.int32(j), t)
        start_scatter(jnp.int32(j), t)
      for j in range(SLAB - NBUF, SLAB):
        wait_scatter(jnp.int32(j), j % NBUF)
      return carry

    lax.fori_loop(0, STAGES, stage_body, 0)

    pltpu.async_copy(feat_hbm.at[srct_v], r0_v.at[pl.ds(0, TAIL)],
                     gsem0).wait()
    pltpu.sync_copy(r0_v.at[pl.ds(0, TAIL)], acc_s.at[dstt_v], add=True)

    plsc.subcore_barrier()
    pltpu.sync_copy(
        acc_s.at[pl.ds(s * ROWS_PER_TILE, ROWS_PER_TILE)],
        out_hbm.at[pl.ds(c * N_NODES + s * ROWS_PER_TILE, ROWS_PER_TILE)])

    @pl.when(s == NS - 1)
    def _():
      pltpu.sync_copy(
          acc_s.at[pl.ds(NS * ROWS_PER_TILE, ROWS_REM)],
          out_hbm.at[pl.ds(c * N_NODES + NS * ROWS_PER_TILE, ROWS_REM)])

  return k(feature, src_main, dst_main, src_tail, dst_tail, zeros)


def _tc_linear_kernel(h0_ref, h1_ref, w_ref, b_ref, o_ref):
  h = h0_ref[...] + h1_ref[...]
  o_ref[...] = lax.dot_general(
      h, w_ref[...], (((1,), (1,)), ((), ())),
      preferred_element_type=jnp.float32) + b_ref[...]


def kernel(feature, edge_index, W, b):
  src = edge_index[0]
  dst = edge_index[1]
  n_main = NW * CH * CHUNK
  src_main = src[:n_main].reshape(NW * STAGES, SLAB, CHUNK)
  dst_main = dst[:n_main].reshape(NW * STAGES, SLAB, CHUNK)
  src_tail = src[n_main:]
  dst_tail = dst[n_main:]
  zeros = jnp.zeros((ROWS_PER_TILE, D), jnp.float32)

  partials = _sc_gather_scatter(feature, src_main, dst_main,
                                src_tail, dst_tail, zeros)
  out = pl.pallas_call(
      _tc_linear_kernel,
      grid=(1,),
      in_specs=[
          pl.BlockSpec((N_NODES, D), lambda i: (0, 0)),
          pl.BlockSpec((N_NODES, D), lambda i: (1, 0)),
          pl.BlockSpec((D, D), lambda i: (0, 0)),
          pl.BlockSpec((1, D), lambda i: (0, 0)),
      ],
      out_specs=pl.BlockSpec((N_NODES, D), lambda i: (0, 0)),
      out_shape=jax.ShapeDtypeStruct((N_NODES, D), jnp.float32),
  )(partials, partials, W, b.reshape(1, D))
  return out


# speedup vs baseline: 1.0164x; 1.0164x over previous
import functools

import jax
import jax.numpy as jnp
from jax import lax
from jax.experimental import pallas as pl
from jax.experimental.pallas import tpu as pltpu
from jax.experimental.pallas import tpu_sc as plsc

N_NODES = 10000
N_EDGES = 320000
D = 128

NC = 2
NS = 16
NW = NC * NS

CHUNK = 96
E_PER_W = N_EDGES // NW
CH = E_PER_W // CHUNK
TAIL = E_PER_W - CH * CHUNK
STAGES = 4
SLAB = CH // STAGES
NBUF = 3
ROWS_PER_TILE = 624
ROWS_REM = N_NODES - NS * ROWS_PER_TILE


def _sc_gather_scatter(feature, src_main, dst_main, src_tail, dst_tail, zeros):
  mesh = plsc.VectorSubcoreMesh(core_axis_name="c", subcore_axis_name="s")

  @functools.partial(
      pl.kernel,
      mesh=mesh,
      out_type=jax.ShapeDtypeStruct((NC * N_NODES, D), jnp.float32),
      scratch_types=[
          pltpu.VMEM((SLAB, CHUNK), jnp.int32),
          pltpu.VMEM((SLAB, CHUNK), jnp.int32),
          pltpu.VMEM((TAIL,), jnp.int32),
          pltpu.VMEM((TAIL,), jnp.int32),
          pltpu.VMEM((CHUNK, D), jnp.float32),
          pltpu.VMEM((CHUNK, D), jnp.float32),
          pltpu.VMEM((CHUNK, D), jnp.float32),
          pltpu.VMEM_SHARED((N_NODES, D), jnp.float32),
          pltpu.SemaphoreType.DMA,
          pltpu.SemaphoreType.DMA,
          pltpu.SemaphoreType.DMA,
          pltpu.SemaphoreType.DMA,
          pltpu.SemaphoreType.DMA,
          pltpu.SemaphoreType.DMA,
          pltpu.SemaphoreType.DMA,
      ],
  )
  def k(feat_hbm, srcm_hbm, dstm_hbm, srct_hbm, dstt_hbm, zero_hbm, out_hbm,
        src_v, dst_v, srct_v, dstt_v, r0_v, r1_v, r2_v, acc_s,
        gsem0, gsem1, gsem2, ssem0, ssem1, ssem2, zsem):
    c = lax.axis_index("c")
    s = lax.axis_index("s")
    wid = s * NC + c

    bufs = (r0_v, r1_v, r2_v)
    gsems = (gsem0, gsem1, gsem2)
    ssems = (ssem0, ssem1, ssem2)

    pltpu.async_copy(zero_hbm,
                     acc_s.at[pl.ds(s * ROWS_PER_TILE, ROWS_PER_TILE)], zsem)

    @pl.when(s == NS - 1)
    def _():
      pltpu.async_copy(zero_hbm.at[pl.ds(0, ROWS_REM)],
                       acc_s.at[pl.ds(NS * ROWS_PER_TILE, ROWS_REM)], zsem)

    pltpu.sync_copy(srct_hbm.at[pl.ds(wid * TAIL, TAIL)], srct_v)
    pltpu.sync_copy(dstt_hbm.at[pl.ds(wid * TAIL, TAIL)], dstt_v)

    def start_gather(j, b):
      pltpu.async_copy(feat_hbm.at[src_v.at[j]], bufs[b], gsems[b])

    def wait_gather(j, b):
      pltpu.make_async_copy(feat_hbm.at[src_v.at[j]], bufs[b],
                            gsems[b]).wait()

    def start_scatter(j, b):
      pltpu.async_copy(bufs[b], acc_s.at[dst_v.at[j]], ssems[b], add=True)

    def wait_scatter(j, b):
      pltpu.make_async_copy(bufs[b], acc_s.at[dst_v.at[j]], ssems[b]).wait()

    def stage_body(st, carry):
      pltpu.sync_copy(srcm_hbm.at[wid * STAGES + st], src_v)
      pltpu.sync_copy(dstm_hbm.at[wid * STAGES + st], dst_v)
      start_gather(0, 0)
      start_gather(1, 1)

      @pl.when(st == 0)
      def _():
        pltpu.make_async_copy(
            zero_hbm, acc_s.at[pl.ds(s * ROWS_PER_TILE, ROWS_PER_TILE)],
            zsem).wait()

        @pl.when(s == NS - 1)
        def _():
          pltpu.make_async_copy(
              zero_hbm.at[pl.ds(0, ROWS_REM)],
              acc_s.at[pl.ds(NS * ROWS_PER_TILE, ROWS_REM)], zsem).wait()

        plsc.subcore_barrier()

      def tri(i, carry2):
        for d in range(NBUF):
          j = NBUF * i + d
          t2 = (d + 2) % NBUF
          wait_gather(j, d)
          start_scatter(j, d)

          @pl.when(j >= 1)
          def _():
            wait_scatter(j - 1, t2)

          start_gather(j + 2, t2)
        return carry2

      lax.fori_loop(0, SLAB // NBUF, tri, 0)
      for j, t in ((SLAB - 2, (SLAB - 2) % NBUF), (SLAB - 1, (SLAB - 1) % NBUF)):
        wait_gather(jnp.int32(j), t)
        start_scatter(jnp.int32(j), t)
      for j in range(SLAB - NBUF, SLAB):
        wait_scatter(jnp.int32(j), j % NBUF)
      return carry

    lax.fori_loop(0, STAGES, stage_body, 0)

    pltpu.async_copy(feat_hbm.at[srct_v], r0_v.at[pl.ds(0, TAIL)],
                     gsem0).wait()
    pltpu.sync_copy(r0_v.at[pl.ds(0, TAIL)], acc_s.at[dstt_v], add=True)

    plsc.subcore_barrier()
    pltpu.sync_copy(
        acc_s.at[pl.ds(s * ROWS_PER_TILE, ROWS_PER_TILE)],
        out_hbm.at[pl.ds(c * N_NODES + s * ROWS_PER_TILE, ROWS_PER_TILE)])

    @pl.when(s == NS - 1)
    def _():
      pltpu.sync_copy(
          acc_s.at[pl.ds(NS * ROWS_PER_TILE, ROWS_REM)],
          out_hbm.at[pl.ds(c * N_NODES + NS * ROWS_PER_TILE, ROWS_REM)])

  return k(feature, src_main, dst_main, src_tail, dst_tail, zeros)


def _tc_linear_kernel(h0_ref, h1_ref, w_ref, b_ref, o_ref):
  h = h0_ref[...] + h1_ref[...]
  o_ref[...] = lax.dot_general(
      h, w_ref[...], (((1,), (1,)), ((), ())),
      preferred_element_type=jnp.float32) + b_ref[...]


def kernel(feature, edge_index, W, b):
  src = edge_index[0]
  dst = edge_index[1]
  n_main = NW * CH * CHUNK
  src_main = src[:n_main].reshape(NW * STAGES, SLAB, CHUNK)
  dst_main = dst[:n_main].reshape(NW * STAGES, SLAB, CHUNK)
  src_tail = src[n_main:]
  dst_tail = dst[n_main:]
  zeros = jnp.zeros((ROWS_PER_TILE, D), jnp.float32)

  partials = _sc_gather_scatter(feature, src_main, dst_main,
                                src_tail, dst_tail, zeros)
  out = pl.pallas_call(
      _tc_linear_kernel,
      grid=(1,),
      in_specs=[
          pl.BlockSpec((N_NODES, D), lambda i: (0, 0)),
          pl.BlockSpec((N_NODES, D), lambda i: (1, 0)),
          pl.BlockSpec((D, D), lambda i: (0, 0)),
          pl.BlockSpec((1, D), lambda i: (0, 0)),
      ],
      out_specs=pl.BlockSpec((N_NODES, D), lambda i: (0, 0)),
      out_shape=jax.ShapeDtypeStruct((N_NODES, D), jnp.float32),
  )(partials, partials, W, b.reshape(1, D))
  return out
